# Initial kernel scaffold; baseline (speedup 1.0000x reference)
import functools

import jax
import jax.numpy as jnp
from jax import lax
from jax.experimental import pallas as pl
from jax.experimental.pallas import tpu as pltpu
from jax.experimental.pallas import tpu_sc as plsc

_N = 10000
_E = 320000
_D = 128
_NC = 2
_NS = 16
_NW = _NC * _NS
_C = 128
_K = 79
_EW = _K * _C
_EPAD = _NW * _EW
_NPAD = 10240
_RPT = _N // _NS

_MESH = dict(core_axis_name="c", subcore_axis_name="s")
_MBLK = 2000


def _sc_degree(dst_pad):
    @functools.partial(
        pl.kernel,
        out_type=jax.ShapeDtypeStruct((_NC, _N, 16), jnp.float32),
        mesh=plsc.VectorSubcoreMesh(**_MESH),
        scratch_types=[
            pltpu.VMEM_SHARED((_NPAD, 16), jnp.float32),
            pltpu.VMEM((_C,), jnp.int32),
            pltpu.VMEM((_C, 16), jnp.float32),
            pltpu.VMEM((_C, 16), jnp.float32),
        ],
    )
    def run(dst_hbm, out_hbm, deg_sh, dst_v, ones_v, zeros_v):
        cid = lax.axis_index("c")
        sid = lax.axis_index("s")

        @pl.loop(0, _C)
        def _(i):
            ones_v[i, pl.ds(0, 16)] = jnp.ones((16,), jnp.float32)
            zeros_v[i, pl.ds(0, 16)] = jnp.zeros((16,), jnp.float32)

        zb = sid * _RPT
        pltpu.sync_copy(zeros_v, deg_sh.at[pl.ds(zb, _C)])
        pltpu.sync_copy(zeros_v, deg_sh.at[pl.ds(zb + 128, _C)])
        pltpu.sync_copy(zeros_v, deg_sh.at[pl.ds(zb + 256, _C)])
        pltpu.sync_copy(zeros_v, deg_sh.at[pl.ds(zb + 384, _C)])
        pltpu.sync_copy(zeros_v.at[pl.ds(0, 113)], deg_sh.at[pl.ds(zb + 512, 113)])
        plsc.subcore_barrier()

        ebase = (cid * _NS + sid) * _EW

        @pl.loop(0, _K)
        def _(k):
            pltpu.sync_copy(dst_hbm.at[pl.ds(ebase + k * _C, _C)], dst_v)
            pltpu.sync_copy(ones_v, deg_sh.at[dst_v], add=True)

        plsc.subcore_barrier()
        pltpu.sync_copy(deg_sh.at[pl.ds(zb, _RPT)],
                        out_hbm.at[cid, pl.ds(zb, _RPT)])

    return run(dst_pad)


def _sc_aggregate(hp, src_pad, dst_pad):
    @functools.partial(
        pl.kernel,
        out_type=jax.ShapeDtypeStruct((_NC, _N, _D), jnp.float32),
        mesh=plsc.VectorSubcoreMesh(**_MESH),
        scratch_types=[
            pltpu.VMEM_SHARED((_NPAD, _D), jnp.float32),
            pltpu.VMEM((_C,), jnp.int32),
            pltpu.VMEM((_C,), jnp.int32),
            pltpu.VMEM((_C, _D), jnp.float32),
        ],
    )
    def run(hp_hbm, src_hbm, dst_hbm, out_hbm, acc_sh, src_v, dst_v, rows_v):
        cid = lax.axis_index("c")
        sid = lax.axis_index("s")

        @pl.loop(0, _C)
        def _(i):
            @pl.loop(0, _D, step=16)
            def _(j):
                rows_v[i, pl.ds(j, 16)] = jnp.zeros((16,), jnp.float32)

        zb = sid * _RPT
        pltpu.sync_copy(rows_v, acc_sh.at[pl.ds(zb, _C)])
        pltpu.sync_copy(rows_v, acc_sh.at[pl.ds(zb + 128, _C)])
        pltpu.sync_copy(rows_v, acc_sh.at[pl.ds(zb + 256, _C)])
        pltpu.sync_copy(rows_v, acc_sh.at[pl.ds(zb + 384, _C)])
        pltpu.sync_copy(rows_v.at[pl.ds(0, 113)], acc_sh.at[pl.ds(zb + 512, 113)])
        plsc.subcore_barrier()

        ebase = (cid * _NS + sid) * _EW

        @pl.loop(0, _K)
        def _(k):
            off = ebase + k * _C
            pltpu.sync_copy(src_hbm.at[pl.ds(off, _C)], src_v)
            pltpu.sync_copy(dst_hbm.at[pl.ds(off, _C)], dst_v)
            pltpu.sync_copy(hp_hbm.at[src_v], rows_v)
            pltpu.sync_copy(rows_v, acc_sh.at[dst_v], add=True)

        plsc.subcore_barrier()
        pltpu.sync_copy(acc_sh.at[pl.ds(zb, _RPT)],
                        out_hbm.at[cid, pl.ds(zb, _RPT)])

    return run(hp, src_pad, dst_pad)


def _dinv_from(deg_ref):
    d = deg_ref[...]
    return lax.rsqrt(d[0, :, 0] + d[1, :, 0] + 1.0)


def _tc1_body(deg_ref, x_ref, w_ref, out_ref):
    dinv = _dinv_from(deg_ref)
    h = jnp.dot(x_ref[...], w_ref[...], preferred_element_type=jnp.float32)
    out_ref[...] = h * dinv[:, None]


def _tc2_body(deg_ref, p_ref, hp_ref, b_ref, w_ref, out_ref):
    dinv = _dinv_from(deg_ref)
    p = p_ref[...]
    s = p[0] + p[1] + hp_ref[...]
    t = jnp.maximum(s * dinv[:, None] + b_ref[...], 0.0)
    h = jnp.dot(t, w_ref[...], preferred_element_type=jnp.float32)
    out_ref[...] = h * dinv[:, None]


def _tc3_body(deg_ref, p_ref, hp_ref, b_ref, w_ref, bfc_ref, out_ref):
    dinv = _dinv_from(deg_ref)
    p = p_ref[...]
    s = p[0] + p[1] + hp_ref[...]
    t = jnp.maximum(s * dinv[:, None] + b_ref[...], 0.0)
    logits = jnp.dot(t, w_ref[...], preferred_element_type=jnp.float32) + bfc_ref[...]
    m = jnp.max(logits, axis=1, keepdims=True)
    lse = jnp.log(jnp.sum(jnp.exp(logits - m), axis=1, keepdims=True)) + m
    out_ref[...] = logits - lse


_DEG_SPEC = pl.BlockSpec((_NC, _MBLK, 16), lambda i: (0, i, 0))
_ROW_SPEC = pl.BlockSpec((_MBLK, _D), lambda i: (i, 0))
_P_SPEC = pl.BlockSpec((_NC, _MBLK, _D), lambda i: (0, i, 0))
_W_SPEC = pl.BlockSpec((_D, _D), lambda i: (0, 0))
_B_SPEC = pl.BlockSpec((1, _D), lambda i: (0, 0))
_GRID = (_N // _MBLK,)
_OUT = jax.ShapeDtypeStruct((_N, _D), jnp.float32)


def _tc1(deg_p, x, w1):
    return pl.pallas_call(
        _tc1_body, grid=_GRID,
        in_specs=[_DEG_SPEC, _ROW_SPEC, _W_SPEC],
        out_specs=_ROW_SPEC, out_shape=_OUT,
    )(deg_p, x, w1)


def _tc2(deg_p, p1, hp, b, w):
    return pl.pallas_call(
        _tc2_body, grid=_GRID,
        in_specs=[_DEG_SPEC, _P_SPEC, _ROW_SPEC, _B_SPEC, _W_SPEC],
        out_specs=_ROW_SPEC, out_shape=_OUT,
    )(deg_p, p1, hp, b, w)


def _tc3(deg_p, p2, hp, b, w, bfc):
    return pl.pallas_call(
        _tc3_body, grid=_GRID,
        in_specs=[_DEG_SPEC, _P_SPEC, _ROW_SPEC, _B_SPEC, _W_SPEC, _B_SPEC],
        out_specs=_ROW_SPEC, out_shape=_OUT,
    )(deg_p, p2, hp, b, w, bfc)


def kernel(x, edge_index, W1, b1, W2, b2, Wfc, bfc):
    pad = _EPAD - _E
    src_pad = jnp.concatenate([edge_index[0], jnp.zeros((pad,), jnp.int32)])
    dst_pad = jnp.concatenate([edge_index[1], jnp.full((pad,), _N, jnp.int32)])
    b1r = b1.reshape(1, _D)
    b2r = b2.reshape(1, _D)
    bfcr = bfc.reshape(1, _D)

    deg_p = _sc_degree(dst_pad)
    h1p = _tc1(deg_p, x, W1)
    p1 = _sc_aggregate(h1p, src_pad, dst_pad)
    h2p = _tc2(deg_p, p1, h1p, b1r, W2)
    p2 = _sc_aggregate(h2p, src_pad, dst_pad)
    return _tc3(deg_p, p2, h2p, b2r, Wfc, bfcr)


# speedup vs baseline: 11.2531x; 11.2531x over previous
import functools

import jax
import jax.numpy as jnp
from jax import lax
from jax.experimental import pallas as pl
from jax.experimental.pallas import tpu as pltpu
from jax.experimental.pallas import tpu_sc as plsc

_N = 10000
_E = 320000
_D = 128
_NC = 2
_NS = 16
_NW = _NC * _NS
_C = 128
_K = 79
_EW = _K * _C
_EPAD = _NW * _EW
_NPAD = 10240
_RPT = _NPAD // _NS

_MESH = dict(core_axis_name="c", subcore_axis_name="s")
_MBLK = 2000


def _sc_degree(dst_pad):
    @functools.partial(
        pl.kernel,
        out_type=jax.ShapeDtypeStruct((_NC, _NPAD, 16), jnp.float32),
        mesh=plsc.VectorSubcoreMesh(**_MESH),
        scratch_types=[
            pltpu.VMEM_SHARED((_NPAD, 16), jnp.float32),
            pltpu.VMEM((_C,), jnp.int32),
            pltpu.VMEM((_C, 16), jnp.float32),
            pltpu.VMEM((_C, 16), jnp.float32),
        ],
    )
    def run(dst_hbm, out_hbm, deg_sh, dst_v, ones_v, zeros_v):
        cid = lax.axis_index("c")
        sid = lax.axis_index("s")

        @pl.loop(0, _C)
        def _(i):
            ones_v[i, pl.ds(0, 16)] = jnp.ones((16,), jnp.float32)
            zeros_v[i, pl.ds(0, 16)] = jnp.zeros((16,), jnp.float32)

        zb = sid * _RPT
        for zo in range(0, _RPT, _C):
            pltpu.sync_copy(zeros_v, deg_sh.at[pl.ds(zb + zo, _C)])
        plsc.subcore_barrier()

        ebase = (cid * _NS + sid) * _EW

        @pl.loop(0, _K)
        def _(k):
            pltpu.sync_copy(dst_hbm.at[pl.ds(ebase + k * _C, _C)], dst_v)
            pltpu.sync_copy(ones_v, deg_sh.at[dst_v], add=True)

        plsc.subcore_barrier()
        pltpu.sync_copy(deg_sh.at[pl.ds(zb, _RPT)],
                        out_hbm.at[cid, pl.ds(zb, _RPT)])

    return run(dst_pad)


def _sc_aggregate(hp, src_pad, dst_pad):
    @functools.partial(
        pl.kernel,
        out_type=jax.ShapeDtypeStruct((_NC, _NPAD, _D), jnp.float32),
        mesh=plsc.VectorSubcoreMesh(**_MESH),
        scratch_types=[
            pltpu.VMEM_SHARED((_NPAD, _D), jnp.float32),
            pltpu.VMEM((_C,), jnp.int32),
            pltpu.VMEM((_C,), jnp.int32),
            pltpu.VMEM((_C, _D), jnp.float32),
        ],
    )
    def run(hp_hbm, src_hbm, dst_hbm, out_hbm, acc_sh, src_v, dst_v, rows_v):
        cid = lax.axis_index("c")
        sid = lax.axis_index("s")

        @pl.loop(0, _C)
        def _(i):
            @pl.loop(0, _D, step=16)
            def _(j):
                rows_v[i, pl.ds(j, 16)] = jnp.zeros((16,), jnp.float32)

        zb = sid * _RPT
        for zo in range(0, _RPT, _C):
            pltpu.sync_copy(rows_v, acc_sh.at[pl.ds(zb + zo, _C)])
        plsc.subcore_barrier()

        ebase = (cid * _NS + sid) * _EW

        @pl.loop(0, _K)
        def _(k):
            off = ebase + k * _C
            pltpu.sync_copy(src_hbm.at[pl.ds(off, _C)], src_v)
            pltpu.sync_copy(dst_hbm.at[pl.ds(off, _C)], dst_v)
            pltpu.sync_copy(hp_hbm.at[src_v], rows_v)
            pltpu.sync_copy(rows_v, acc_sh.at[dst_v], add=True)

        plsc.subcore_barrier()
        pltpu.sync_copy(acc_sh.at[pl.ds(zb, _RPT)],
                        out_hbm.at[cid, pl.ds(zb, _RPT)])

    return run(hp, src_pad, dst_pad)


def _dinv_from(deg_ref):
    d = deg_ref[...]
    return lax.rsqrt(d[0, :, 0] + d[1, :, 0] + 1.0)


def _tc1_body(deg_ref, x_ref, w_ref, out_ref):
    dinv = _dinv_from(deg_ref)
    h = jnp.dot(x_ref[...], w_ref[...], preferred_element_type=jnp.float32)
    out_ref[...] = h * dinv[:, None]


def _tc2_body(deg_ref, p_ref, hp_ref, b_ref, w_ref, out_ref):
    dinv = _dinv_from(deg_ref)
    p = p_ref[...]
    s = p[0] + p[1] + hp_ref[...]
    t = jnp.maximum(s * dinv[:, None] + b_ref[...], 0.0)
    h = jnp.dot(t, w_ref[...], preferred_element_type=jnp.float32)
    out_ref[...] = h * dinv[:, None]


def _tc3_body(deg_ref, p_ref, hp_ref, b_ref, w_ref, bfc_ref, out_ref):
    dinv = _dinv_from(deg_ref)
    p = p_ref[...]
    s = p[0] + p[1] + hp_ref[...]
    t = jnp.maximum(s * dinv[:, None] + b_ref[...], 0.0)
    logits = jnp.dot(t, w_ref[...], preferred_element_type=jnp.float32) + bfc_ref[...]
    m = jnp.max(logits, axis=1, keepdims=True)
    lse = jnp.log(jnp.sum(jnp.exp(logits - m), axis=1, keepdims=True)) + m
    out_ref[...] = logits - lse


_DEG_SPEC = pl.BlockSpec((_NC, _MBLK, 16), lambda i: (0, i, 0))
_ROW_SPEC = pl.BlockSpec((_MBLK, _D), lambda i: (i, 0))
_P_SPEC = pl.BlockSpec((_NC, _MBLK, _D), lambda i: (0, i, 0))
_W_SPEC = pl.BlockSpec((_D, _D), lambda i: (0, 0))
_B_SPEC = pl.BlockSpec((1, _D), lambda i: (0, 0))
_GRID = (_N // _MBLK,)
_OUT = jax.ShapeDtypeStruct((_N, _D), jnp.float32)


def _tc1(deg_p, x, w1):
    return pl.pallas_call(
        _tc1_body, grid=_GRID,
        in_specs=[_DEG_SPEC, _ROW_SPEC, _W_SPEC],
        out_specs=_ROW_SPEC, out_shape=_OUT,
    )(deg_p, x, w1)


def _tc2(deg_p, p1, hp, b, w):
    return pl.pallas_call(
        _tc2_body, grid=_GRID,
        in_specs=[_DEG_SPEC, _P_SPEC, _ROW_SPEC, _B_SPEC, _W_SPEC],
        out_specs=_ROW_SPEC, out_shape=_OUT,
    )(deg_p, p1, hp, b, w)


def _tc3(deg_p, p2, hp, b, w, bfc):
    return pl.pallas_call(
        _tc3_body, grid=_GRID,
        in_specs=[_DEG_SPEC, _P_SPEC, _ROW_SPEC, _B_SPEC, _W_SPEC, _B_SPEC],
        out_specs=_ROW_SPEC, out_shape=_OUT,
    )(deg_p, p2, hp, b, w, bfc)


def kernel(x, edge_index, W1, b1, W2, b2, Wfc, bfc):
    pad = _EPAD - _E
    src_pad = jnp.concatenate([edge_index[0], jnp.zeros((pad,), jnp.int32)])
    dst_pad = jnp.concatenate([edge_index[1], jnp.full((pad,), _N, jnp.int32)])
    b1r = b1.reshape(1, _D)
    b2r = b2.reshape(1, _D)
    bfcr = bfc.reshape(1, _D)

    deg_p = _sc_degree(dst_pad)
    h1p = _tc1(deg_p, x, W1)
    p1 = _sc_aggregate(h1p, src_pad, dst_pad)
    h2p = _tc2(deg_p, p1, h1p, b1r, W2)
    p2 = _sc_aggregate(h2p, src_pad, dst_pad)
    return _tc3(deg_p, p2, h2p, b2r, Wfc, bfcr)
